# Initial kernel scaffold; baseline (speedup 1.0000x reference)
import jax
import jax.numpy as jnp
from jax.experimental import pallas as pl


def kernel(tokens, table):
    raise NotImplementedError("write your pallas kernel here")


# speedup vs baseline: 5.9709x; 5.9709x over previous
import functools

import jax
import jax.numpy as jnp
from jax import lax
from jax.experimental import pallas as pl
from jax.experimental.pallas import tpu as pltpu
from jax.experimental.pallas import tpu_sc as plsc

VOCAB = 100000
D = 128
NC, NS = 2, 16
NW = NC * NS
K = 128


def _emb_call(n, b_per_w, n_chunks):
    mesh = plsc.VectorSubcoreMesh(
        core_axis_name="c", subcore_axis_name="s",
        num_cores=NC, num_subcores=NS,
    )

    @functools.partial(
        pl.kernel,
        out_type=jax.ShapeDtypeStruct((n, D), jnp.float32),
        mesh=mesh,
        scratch_types=[
            pltpu.VMEM((b_per_w,), jnp.int32),
            pltpu.VMEM((K, D), jnp.float32),
            pltpu.SemaphoreType.DMA,
        ],
    )
    def emb(tab_hbm, idx_hbm, out_hbm, idx_v, rows_v, sem):
        wid = lax.axis_index("s") * NC + lax.axis_index("c")
        base = wid * b_per_w
        pltpu.sync_copy(idx_hbm.at[pl.ds(base, b_per_w)], idx_v)

        def body(g, carry):
            off = g * K
            pltpu.async_copy(
                tab_hbm.at[idx_v.at[pl.ds(off, K)]], rows_v, sem
            ).wait()
            pltpu.sync_copy(rows_v, out_hbm.at[pl.ds(base + off, K)])
            return carry

        lax.fori_loop(0, n_chunks, body, 0)

    return emb


def kernel(tokens, table):
    bsz, seq = tokens.shape
    n = bsz * seq
    b_per_w = n // NW
    n_chunks = b_per_w // K

    idx = jnp.where(tokens == 0, VOCAB, tokens).reshape(-1).astype(jnp.int32)
    tab = jnp.concatenate([table, jnp.zeros((8, D), table.dtype)], axis=0)

    out = _emb_call(n, b_per_w, n_chunks)(tab, idx)
    return out.reshape(bsz, seq, D)


# speedup vs baseline: 8.3343x; 1.3958x over previous
import functools

import jax
import jax.numpy as jnp
from jax import lax
from jax.experimental import pallas as pl
from jax.experimental.pallas import tpu as pltpu
from jax.experimental.pallas import tpu_sc as plsc

VOCAB = 100000
D = 128
NC, NS = 2, 16
NW = NC * NS
K = 128
NBUF = 4


def _emb_call(n, b_per_w, n_chunks):
    mesh = plsc.VectorSubcoreMesh(
        core_axis_name="c", subcore_axis_name="s",
        num_cores=NC, num_subcores=NS,
    )
    n_outer = n_chunks // NBUF

    @functools.partial(
        pl.kernel,
        out_type=jax.ShapeDtypeStruct((n, D), jnp.float32),
        mesh=mesh,
        scratch_types=(
            [pltpu.VMEM((b_per_w,), jnp.int32)]
            + [pltpu.VMEM((K, D), jnp.float32) for _ in range(NBUF)]
            + [pltpu.SemaphoreType.DMA for _ in range(2 * NBUF)]
        ),
    )
    def emb(tab_hbm, idx_hbm, out_hbm, idx_v, *bufs_and_sems):
        bufs = bufs_and_sems[:NBUF]
        gsem = bufs_and_sems[NBUF:2 * NBUF]
        wsem = bufs_and_sems[2 * NBUF:]
        wid = lax.axis_index("s") * NC + lax.axis_index("c")
        base = wid * b_per_w
        pltpu.sync_copy(idx_hbm.at[pl.ds(base, b_per_w)], idx_v)

        def gissue(c, b):
            pltpu.async_copy(
                tab_hbm.at[idx_v.at[pl.ds(c * K, K)]], bufs[b], gsem[b])

        def gwait(b):
            pltpu.make_async_copy(
                tab_hbm.at[pl.ds(0, K)], bufs[b], gsem[b]).wait()

        def wissue(c, b):
            pltpu.async_copy(
                bufs[b], out_hbm.at[pl.ds(base + c * K, K)], wsem[b])

        def wwait(b):
            pltpu.make_async_copy(
                bufs[b], out_hbm.at[pl.ds(base, K)], wsem[b]).wait()

        def stage(c, b, prime_c, need_wwait):
            gwait(b)
            wissue(c, b)
            if prime_c is not None:
                pb = (b + 2) % NBUF
                if need_wwait:
                    wwait(pb)
                gissue(prime_c, pb)

        gissue(0, 0)
        gissue(1, 1)

        stage(0, 0, 2, False)
        stage(1, 1, 3, False)
        stage(2, 2, 4, True)
        stage(3, 3, 5, True)

        def body(t, carry):
            for b in range(NBUF):
                c = t * NBUF + b
                stage(c, b, c + 2, True)
            return carry

        lax.fori_loop(1, n_outer - 1, body, 0)

        c0 = (n_outer - 1) * NBUF
        stage(c0 + 0, 0, c0 + 2, True)
        stage(c0 + 1, 1, c0 + 3, True)
        stage(c0 + 2, 2, None, False)
        stage(c0 + 3, 3, None, False)
        for b in range(NBUF):
            wwait(b)

    return emb


def kernel(tokens, table):
    bsz, seq = tokens.shape
    n = bsz * seq
    b_per_w = n // NW
    n_chunks = b_per_w // K
    assert n % NW == 0 and b_per_w % (NBUF * K) == 0

    idx = jnp.where(tokens == 0, VOCAB, tokens).reshape(-1).astype(jnp.int32)
    tab = jnp.concatenate([table, jnp.zeros((8, D), table.dtype)], axis=0)

    out = _emb_call(n, b_per_w, n_chunks)(tab, idx)
    return out.reshape(bsz, seq, D)


# speedup vs baseline: 9.1216x; 1.0945x over previous
import functools

import jax
import jax.numpy as jnp
from jax import lax
from jax.experimental import pallas as pl
from jax.experimental.pallas import tpu as pltpu
from jax.experimental.pallas import tpu_sc as plsc

VOCAB = 100000
D = 128
NC, NS = 2, 16
NW = NC * NS
K = 128
NBUF = 4


def _emb_call(n, b_per_w, n_chunks):
    mesh = plsc.VectorSubcoreMesh(
        core_axis_name="c", subcore_axis_name="s",
        num_cores=NC, num_subcores=NS,
    )
    n_outer = n_chunks // NBUF

    @functools.partial(
        pl.kernel,
        out_type=jax.ShapeDtypeStruct((n, D), jnp.float32),
        mesh=mesh,
        scratch_types=(
            [pltpu.VMEM((b_per_w,), jnp.int32)]
            + [pltpu.VMEM((K, D), jnp.float32) for _ in range(NBUF)]
            + [pltpu.SemaphoreType.DMA for _ in range(2 * NBUF)]
        ),
    )
    def emb(tab_hbm, idx_hbm, out_hbm, idx_v, *bufs_and_sems):
        bufs = bufs_and_sems[:NBUF]
        gsem = bufs_and_sems[NBUF:2 * NBUF]
        wsem = bufs_and_sems[2 * NBUF:]
        wid = lax.axis_index("s") * NC + lax.axis_index("c")
        base = wid * b_per_w
        pltpu.sync_copy(idx_hbm.at[pl.ds(base, b_per_w)], idx_v)

        def gissue(c, b):
            pltpu.async_copy(
                tab_hbm.at[idx_v.at[pl.ds(c * K, K)]], bufs[b], gsem[b])

        def gwait(b):
            pltpu.make_async_copy(
                tab_hbm.at[pl.ds(0, K)], bufs[b], gsem[b]).wait()

        def wissue(c, b):
            pltpu.async_copy(
                bufs[b], out_hbm.at[pl.ds(base + c * K, K)], wsem[b])

        def wwait(b):
            pltpu.make_async_copy(
                bufs[b], out_hbm.at[pl.ds(base, K)], wsem[b]).wait()

        def fix_padding(c, b):
            vecs = [idx_v[pl.ds(c * K + j * 16, 16)] for j in range(K // 16)]
            vmin = vecs[0]
            for v in vecs[1:]:
                vmin = jnp.minimum(vmin, v)
            min_tok = vmin[0]
            for l in range(1, 16):
                min_tok = jnp.minimum(min_tok, vmin[l])
            zeros16 = jnp.zeros((16,), jnp.float32)

            @pl.when(min_tok == 0)
            def _():
                def grp_body(j, carry):
                    v = idx_v[pl.ds(c * K + j * 16, 16)]
                    for l in range(16):
                        @pl.when(v[l] == 0)
                        def _zero_row():
                            row = j * 16 + l
                            for cc in range(D // 16):
                                bufs[b][row, pl.ds(cc * 16, 16)] = zeros16
                    return carry
                lax.fori_loop(0, K // 16, grp_body, 0)

        def stage(c, b, prime_c, need_wwait):
            gwait(b)
            fix_padding(c, b)
            wissue(c, b)
            if prime_c is not None:
                pb = (b + 2) % NBUF
                if need_wwait:
                    wwait(pb)
                gissue(prime_c, pb)

        gissue(0, 0)
        gissue(1, 1)

        stage(0, 0, 2, False)
        stage(1, 1, 3, False)
        stage(2, 2, 4, True)
        stage(3, 3, 5, True)

        def body(t, carry):
            for b in range(NBUF):
                c = t * NBUF + b
                stage(c, b, c + 2, True)
            return carry

        lax.fori_loop(1, n_outer - 1, body, 0)

        c0 = (n_outer - 1) * NBUF
        stage(c0 + 0, 0, c0 + 2, True)
        stage(c0 + 1, 1, c0 + 3, True)
        stage(c0 + 2, 2, None, False)
        stage(c0 + 3, 3, None, False)
        for b in range(NBUF):
            wwait(b)

    return emb


def kernel(tokens, table):
    bsz, seq = tokens.shape
    n = bsz * seq
    b_per_w = n // NW
    n_chunks = b_per_w // K
    assert n % NW == 0 and b_per_w % (NBUF * K) == 0

    idx = tokens.reshape(-1).astype(jnp.int32)
    out = _emb_call(n, b_per_w, n_chunks)(table, idx)
    return out.reshape(bsz, seq, D)
